# baseline (speedup 1.0000x reference)
import functools

import jax
import jax.numpy as jnp
from jax import lax
from jax.experimental import pallas as pl
from jax.experimental.pallas import tpu as pltpu
from jax.experimental.pallas import tpu_sc as plsc

N = 10000
E = 320000
D_IN = 128
D_HID = 16
D_OUT = 2

NC = 2
NS = 16
NW = NC * NS
CH = 128
EW = E // NW
NCHUNK = 80
EW_PAD = NCHUNK * CH
NBUF = 4
NGROUP = NCHUNK // NBUF
N_PAD = 10112
RZ = N_PAD // NS
WD = 8

_mesh = functools.partial(
    pl.kernel,
    mesh=plsc.VectorSubcoreMesh(core_axis_name="c", subcore_axis_name="s"),
    compiler_params=pltpu.CompilerParams(use_tc_tiling_on_sc=False),
)


def _sc_agg_body(hs_hbm, srcp_hbm, dstp_hbm, z_hbm, out_hbm,
                 src_v, dst_v, acc_sp,
                 rows0, rows1, rows2, rows3,
                 g0, g1, g2, g3, s0, s1, s2, s3):
    rows = (rows0, rows1, rows2, rows3)
    gsem = (g0, g1, g2, g3)
    ssem = (s0, s1, s2, s3)
    cid = lax.axis_index("c")
    sid = lax.axis_index("s")
    wid = cid * NS + sid
    pltpu.sync_copy(srcp_hbm.at[wid], src_v)
    pltpu.sync_copy(dstp_hbm.at[wid], dst_v)
    pltpu.sync_copy(z_hbm.at[pl.ds(sid * RZ, RZ)],
                    acc_sp.at[pl.ds(sid * RZ, RZ)])
    plsc.subcore_barrier()

    for b in range(NBUF):
        pltpu.async_copy(hs_hbm.at[src_v.at[b]], rows[b], gsem[b])

    def group(p, carry):
        j0 = p * NBUF
        for b in range(NBUF):
            pltpu.make_async_copy(hs_hbm.at[src_v.at[j0 + b]],
                                  rows[b], gsem[b]).wait()
            pltpu.async_copy(rows[b], acc_sp.at[dst_v.at[j0 + b]],
                             ssem[b], add=True)
        for b in range(NBUF):
            jn = jnp.minimum(j0 + b + NBUF, NCHUNK - 1)
            pltpu.make_async_copy(rows[b], acc_sp.at[dst_v.at[j0 + b]],
                                  ssem[b]).wait()
            pltpu.async_copy(hs_hbm.at[src_v.at[jn]], rows[b], gsem[b])
        return carry

    lax.fori_loop(0, NGROUP, group, 0)
    for b in range(NBUF):
        pltpu.make_async_copy(hs_hbm.at[src_v.at[NCHUNK - 1]],
                              rows[b], gsem[b]).wait()
    plsc.subcore_barrier()
    pltpu.sync_copy(acc_sp.at[pl.ds(sid * RZ, RZ)],
                    out_hbm.at[cid, pl.ds(sid * RZ, RZ)])


_sc_agg = _mesh(
    _sc_agg_body,
    out_type=jax.ShapeDtypeStruct((NC, N_PAD, D_HID), jnp.float32),
    scratch_types=(
        [pltpu.VMEM((NCHUNK, CH), jnp.int32),
         pltpu.VMEM((NCHUNK, CH), jnp.int32),
         pltpu.VMEM_SHARED((N_PAD, D_HID), jnp.float32)]
        + [pltpu.VMEM((CH, D_HID), jnp.float32)] * NBUF
        + [pltpu.SemaphoreType.DMA] * (2 * NBUF)
    ),
)


def _sc_deg_body(dstp_hbm, ones_hbm, z_hbm, out_hbm,
                 dst_v, ones_v, deg_sp):
    cid = lax.axis_index("c")
    sid = lax.axis_index("s")
    wid = cid * NS + sid
    pltpu.sync_copy(dstp_hbm.at[wid], dst_v)
    pltpu.sync_copy(ones_hbm, ones_v)
    pltpu.sync_copy(z_hbm.at[pl.ds(sid * RZ, RZ)],
                    deg_sp.at[pl.ds(sid * RZ, RZ)])
    plsc.subcore_barrier()

    def chunk(j, carry):
        pltpu.sync_copy(ones_v, deg_sp.at[dst_v.at[j]], add=True)
        return carry

    lax.fori_loop(0, NCHUNK, chunk, 0)
    plsc.subcore_barrier()
    pltpu.sync_copy(deg_sp.at[pl.ds(sid * RZ, RZ)],
                    out_hbm.at[cid, pl.ds(sid * RZ, RZ)])


_sc_deg = _mesh(
    _sc_deg_body,
    out_type=jax.ShapeDtypeStruct((NC, N_PAD, WD), jnp.float32),
    scratch_types=[
        pltpu.VMEM((NCHUNK, CH), jnp.int32),
        pltpu.VMEM((CH, WD), jnp.float32),
        pltpu.VMEM_SHARED((N_PAD, WD), jnp.float32),
    ],
)


def _tc1_body(deg_ref, x_ref, w1_ref, dinv_ref, hs1_ref):
    deg = 1.0 + deg_ref[0, :N, 0:1] + deg_ref[1, :N, 0:1]
    dinv = lax.rsqrt(deg)
    h1 = jnp.dot(x_ref[...], w1_ref[...],
                 preferred_element_type=jnp.float32)
    hs1_ref[...] = h1 * dinv
    dinv_ref[...] = jnp.broadcast_to(dinv, (N, D_HID))


_tc1 = pl.pallas_call(
    _tc1_body,
    out_shape=(
        jax.ShapeDtypeStruct((N, D_HID), jnp.float32),
        jax.ShapeDtypeStruct((N, D_HID), jnp.float32),
    ),
)


def _tc2_body(parts_ref, hs1_ref, dinv_ref, b1_ref, hs2_ref):
    agg = parts_ref[0, :N] + parts_ref[1, :N] + hs1_ref[...]
    z1 = jnp.maximum(dinv_ref[...] * agg + b1_ref[...], 0.0)
    hs2_ref[...] = z1 * dinv_ref[...]


_tc2 = pl.pallas_call(
    _tc2_body,
    out_shape=jax.ShapeDtypeStruct((N, D_HID), jnp.float32),
)


def _tc3_body(parts_ref, hs2_ref, dinv_ref, w2_ref, b2_ref, out_ref):
    agg = parts_ref[0, :N] + parts_ref[1, :N] + hs2_ref[...]
    pre = dinv_ref[...] * agg
    logits = jnp.dot(pre, w2_ref[...],
                     preferred_element_type=jnp.float32) + b2_ref[...]
    m = jnp.max(logits, axis=1, keepdims=True)
    lse = m + jnp.log(jnp.sum(jnp.exp(logits - m), axis=1, keepdims=True))
    out_ref[...] = logits - lse


_tc3 = pl.pallas_call(
    _tc3_body,
    out_shape=jax.ShapeDtypeStruct((N, D_OUT), jnp.float32),
)


def kernel(x, edge_index, W1, b1, W2, b2):
    src = edge_index[0].astype(jnp.int32)
    dst = edge_index[1].astype(jnp.int32)
    pad = EW_PAD - EW
    srcp = jnp.pad(src.reshape(NW, EW), ((0, 0), (0, pad)),
                   constant_values=0).reshape(NW, NCHUNK, CH)
    dstp = jnp.pad(dst.reshape(NW, EW), ((0, 0), (0, pad)),
                   constant_values=N).reshape(NW, NCHUNK, CH)
    zeros16 = jnp.zeros((N_PAD, D_HID), jnp.float32)
    zeros_d = jnp.zeros((N_PAD, WD), jnp.float32)
    ones_d = jnp.ones((CH, WD), jnp.float32)

    deg_parts = _sc_deg(dstp, ones_d, zeros_d)
    dinv, hs1 = _tc1(deg_parts, x, W1)
    agg1 = _sc_agg(hs1, srcp, dstp, zeros16)
    hs2 = _tc2(agg1, hs1, dinv, b1)
    agg2 = _sc_agg(hs2, srcp, dstp, zeros16)
    return _tc3(agg2, hs2, dinv, W2, b2)


# speedup vs baseline: 1.2788x; 1.2788x over previous
import functools

import jax
import jax.numpy as jnp
from jax import lax
from jax.experimental import pallas as pl
from jax.experimental.pallas import tpu as pltpu
from jax.experimental.pallas import tpu_sc as plsc

N = 10000
E = 320000
D_IN = 128
D_HID = 16
D_OUT = 2

NC = 2
NS = 16
NW = NC * NS
CH = 128
EW = E // NW
NCHUNK = 80
EW_PAD = NCHUNK * CH
NBUF = 4
NGROUP = NCHUNK // NBUF
N_PAD = 10112
RZ = N_PAD // NS
WD = 8

_mesh = functools.partial(
    pl.kernel,
    mesh=plsc.VectorSubcoreMesh(core_axis_name="c", subcore_axis_name="s"),
    compiler_params=pltpu.CompilerParams(use_tc_tiling_on_sc=False),
)


def _sc_agg_body(hs_hbm, srcp_hbm, dstp_hbm, z_hbm, out_hbm,
                 src_v, dst_v, acc_sp, rows0, rows1, g0, g1):
    cid = lax.axis_index("c")
    sid = lax.axis_index("s")
    wid = cid * NS + sid
    pltpu.sync_copy(srcp_hbm.at[wid], src_v)
    pltpu.sync_copy(dstp_hbm.at[wid], dst_v)
    pltpu.sync_copy(z_hbm.at[pl.ds(sid * RZ, RZ)],
                    acc_sp.at[pl.ds(sid * RZ, RZ)])
    plsc.subcore_barrier()

    pltpu.async_copy(hs_hbm.at[src_v.at[0]], rows0, g0)

    def pair(p, carry):
        j = 2 * p
        pltpu.async_copy(hs_hbm.at[src_v.at[j + 1]], rows1, g1)
        pltpu.make_async_copy(hs_hbm.at[src_v.at[j]], rows0, g0).wait()
        pltpu.sync_copy(rows0, acc_sp.at[dst_v.at[j]], add=True)
        jn = jnp.minimum(j + 2, NCHUNK - 1)
        pltpu.async_copy(hs_hbm.at[src_v.at[jn]], rows0, g0)
        pltpu.make_async_copy(hs_hbm.at[src_v.at[j + 1]], rows1, g1).wait()
        pltpu.sync_copy(rows1, acc_sp.at[dst_v.at[j + 1]], add=True)
        return carry

    lax.fori_loop(0, NCHUNK // 2, pair, 0)
    pltpu.make_async_copy(hs_hbm.at[src_v.at[NCHUNK - 1]], rows0, g0).wait()
    plsc.subcore_barrier()
    pltpu.sync_copy(acc_sp.at[pl.ds(sid * RZ, RZ)],
                    out_hbm.at[cid, pl.ds(sid * RZ, RZ)])


_sc_agg = _mesh(
    _sc_agg_body,
    out_type=jax.ShapeDtypeStruct((NC, N_PAD, D_HID), jnp.float32),
    scratch_types=(
        [pltpu.VMEM((NCHUNK, CH), jnp.int32),
         pltpu.VMEM((NCHUNK, CH), jnp.int32),
         pltpu.VMEM_SHARED((N_PAD, D_HID), jnp.float32)]
        + [pltpu.VMEM((CH, D_HID), jnp.float32)] * 2
        + [pltpu.SemaphoreType.DMA] * 2
    ),
)


def _sc_deg_body(dstp_hbm, ones_hbm, z_hbm, out_hbm,
                 dst_v, ones_v, deg_sp):
    cid = lax.axis_index("c")
    sid = lax.axis_index("s")
    wid = cid * NS + sid
    pltpu.sync_copy(dstp_hbm.at[wid], dst_v)
    pltpu.sync_copy(ones_hbm, ones_v)
    pltpu.sync_copy(z_hbm.at[pl.ds(sid * RZ, RZ)],
                    deg_sp.at[pl.ds(sid * RZ, RZ)])
    plsc.subcore_barrier()

    def chunk(j, carry):
        pltpu.sync_copy(ones_v, deg_sp.at[dst_v.at[j]], add=True)
        return carry

    lax.fori_loop(0, NCHUNK, chunk, 0)
    plsc.subcore_barrier()
    pltpu.sync_copy(deg_sp.at[pl.ds(sid * RZ, RZ)],
                    out_hbm.at[cid, pl.ds(sid * RZ, RZ)])


_sc_deg = _mesh(
    _sc_deg_body,
    out_type=jax.ShapeDtypeStruct((NC, N_PAD, WD), jnp.float32),
    scratch_types=[
        pltpu.VMEM((NCHUNK, CH), jnp.int32),
        pltpu.VMEM((CH, WD), jnp.float32),
        pltpu.VMEM_SHARED((N_PAD, WD), jnp.float32),
    ],
)


def _tc1_body(deg_ref, x_ref, w1_ref, dinv_ref, hs1_ref):
    deg = 1.0 + deg_ref[0, :N, 0:1] + deg_ref[1, :N, 0:1]
    dinv = lax.rsqrt(deg)
    h1 = jnp.dot(x_ref[...], w1_ref[...],
                 preferred_element_type=jnp.float32)
    hs1_ref[...] = h1 * dinv
    dinv_ref[...] = jnp.broadcast_to(dinv, (N, D_HID))


_tc1 = pl.pallas_call(
    _tc1_body,
    out_shape=(
        jax.ShapeDtypeStruct((N, D_HID), jnp.float32),
        jax.ShapeDtypeStruct((N, D_HID), jnp.float32),
    ),
)


def _tc2_body(parts_ref, hs1_ref, dinv_ref, b1_ref, hs2_ref):
    agg = parts_ref[0, :N] + parts_ref[1, :N] + hs1_ref[...]
    z1 = jnp.maximum(dinv_ref[...] * agg + b1_ref[...], 0.0)
    hs2_ref[...] = z1 * dinv_ref[...]


_tc2 = pl.pallas_call(
    _tc2_body,
    out_shape=jax.ShapeDtypeStruct((N, D_HID), jnp.float32),
)


def _tc3_body(parts_ref, hs2_ref, dinv_ref, w2_ref, b2_ref, out_ref):
    agg = parts_ref[0, :N] + parts_ref[1, :N] + hs2_ref[...]
    pre = dinv_ref[...] * agg
    logits = jnp.dot(pre, w2_ref[...],
                     preferred_element_type=jnp.float32) + b2_ref[...]
    m = jnp.max(logits, axis=1, keepdims=True)
    lse = m + jnp.log(jnp.sum(jnp.exp(logits - m), axis=1, keepdims=True))
    out_ref[...] = logits - lse


_tc3 = pl.pallas_call(
    _tc3_body,
    out_shape=jax.ShapeDtypeStruct((N, D_OUT), jnp.float32),
)


def kernel(x, edge_index, W1, b1, W2, b2):
    src = edge_index[0].astype(jnp.int32)
    dst = edge_index[1].astype(jnp.int32)
    pad = EW_PAD - EW
    srcp = jnp.pad(src.reshape(NW, EW), ((0, 0), (0, pad)),
                   constant_values=0).reshape(NW, NCHUNK, CH)
    dstp = jnp.pad(dst.reshape(NW, EW), ((0, 0), (0, pad)),
                   constant_values=N).reshape(NW, NCHUNK, CH)
    zeros16 = jnp.zeros((N_PAD, D_HID), jnp.float32)
    zeros_d = jnp.zeros((N_PAD, WD), jnp.float32)
    ones_d = jnp.ones((CH, WD), jnp.float32)

    deg_parts = _sc_deg(dstp, ones_d, zeros_d)
    dinv, hs1 = _tc1(deg_parts, x, W1)
    agg1 = _sc_agg(hs1, srcp, dstp, zeros16)
    hs2 = _tc2(agg1, hs1, dinv, b1)
    agg2 = _sc_agg(hs2, srcp, dstp, zeros16)
    return _tc3(agg2, hs2, dinv, W2, b2)


# speedup vs baseline: 1.6165x; 1.2641x over previous
import functools

import jax
import jax.numpy as jnp
from jax import lax
from jax.experimental import pallas as pl
from jax.experimental.pallas import tpu as pltpu
from jax.experimental.pallas import tpu_sc as plsc

N = 10000
E = 320000
D_IN = 128
D_HID = 16
D_OUT = 2

NC = 2
NS = 16
NW = NC * NS
CH = 128
EW = E // NW
NCHUNK = 80
EW_PAD = NCHUNK * CH
CB = 2048
NB = EW_PAD // CB
N_PAD = 10112
RZ = N_PAD // NS
WD = 8

_mesh = functools.partial(
    pl.kernel,
    mesh=plsc.VectorSubcoreMesh(core_axis_name="c", subcore_axis_name="s"),
    compiler_params=pltpu.CompilerParams(use_tc_tiling_on_sc=False),
)


def _sc_agg_body(hs_hbm, srcp_hbm, dstp_hbm, z_hbm, out_hbm,
                 src_v, dst_v, acc_sp, rows_v, sem):
    cid = lax.axis_index("c")
    sid = lax.axis_index("s")
    wid = cid * NS + sid
    pltpu.sync_copy(srcp_hbm.at[wid], src_v)
    pltpu.sync_copy(dstp_hbm.at[wid], dst_v)
    pltpu.sync_copy(z_hbm.at[pl.ds(sid * RZ, RZ)],
                    acc_sp.at[pl.ds(sid * RZ, RZ)])
    plsc.subcore_barrier()

    def chunk(m, carry):
        pltpu.async_copy(hs_hbm.at[src_v.at[m]], rows_v, sem).wait()
        pltpu.sync_copy(rows_v, acc_sp.at[dst_v.at[m]], add=True)
        return carry

    lax.fori_loop(0, NB, chunk, 0)
    plsc.subcore_barrier()
    pltpu.sync_copy(acc_sp.at[pl.ds(sid * RZ, RZ)],
                    out_hbm.at[cid, pl.ds(sid * RZ, RZ)])


_sc_agg = _mesh(
    _sc_agg_body,
    out_type=jax.ShapeDtypeStruct((NC, N_PAD, D_HID), jnp.float32),
    scratch_types=(
        [pltpu.VMEM((NB, CB), jnp.int32),
         pltpu.VMEM((NB, CB), jnp.int32),
         pltpu.VMEM_SHARED((N_PAD, D_HID), jnp.float32)]
        + [pltpu.VMEM((CB, D_HID), jnp.float32),
           pltpu.SemaphoreType.DMA]
    ),
)


def _sc_deg_body(dstp_hbm, ones_hbm, z_hbm, out_hbm,
                 dst_v, ones_v, deg_sp):
    cid = lax.axis_index("c")
    sid = lax.axis_index("s")
    wid = cid * NS + sid
    pltpu.sync_copy(dstp_hbm.at[wid], dst_v)
    pltpu.sync_copy(ones_hbm, ones_v)
    pltpu.sync_copy(z_hbm.at[pl.ds(sid * RZ, RZ)],
                    deg_sp.at[pl.ds(sid * RZ, RZ)])
    plsc.subcore_barrier()

    def chunk(j, carry):
        pltpu.sync_copy(ones_v, deg_sp.at[dst_v.at[j]], add=True)
        return carry

    lax.fori_loop(0, NB, chunk, 0)
    plsc.subcore_barrier()
    pltpu.sync_copy(deg_sp.at[pl.ds(sid * RZ, RZ)],
                    out_hbm.at[cid, pl.ds(sid * RZ, RZ)])


_sc_deg = _mesh(
    _sc_deg_body,
    out_type=jax.ShapeDtypeStruct((NC, N_PAD, WD), jnp.float32),
    scratch_types=[
        pltpu.VMEM((NB, CB), jnp.int32),
        pltpu.VMEM((CB, WD), jnp.float32),
        pltpu.VMEM_SHARED((N_PAD, WD), jnp.float32),
    ],
)


def _tc1_body(deg_ref, x_ref, w1_ref, dinv_ref, hs1_ref):
    deg = 1.0 + deg_ref[0, :N, 0:1] + deg_ref[1, :N, 0:1]
    dinv = lax.rsqrt(deg)
    h1 = jnp.dot(x_ref[...], w1_ref[...],
                 preferred_element_type=jnp.float32)
    hs1_ref[...] = h1 * dinv
    dinv_ref[...] = jnp.broadcast_to(dinv, (N, D_HID))


_tc1 = pl.pallas_call(
    _tc1_body,
    out_shape=(
        jax.ShapeDtypeStruct((N, D_HID), jnp.float32),
        jax.ShapeDtypeStruct((N, D_HID), jnp.float32),
    ),
)


def _tc2_body(parts_ref, hs1_ref, dinv_ref, b1_ref, hs2_ref):
    agg = parts_ref[0, :N] + parts_ref[1, :N] + hs1_ref[...]
    z1 = jnp.maximum(dinv_ref[...] * agg + b1_ref[...], 0.0)
    hs2_ref[...] = z1 * dinv_ref[...]


_tc2 = pl.pallas_call(
    _tc2_body,
    out_shape=jax.ShapeDtypeStruct((N, D_HID), jnp.float32),
)


def _tc3_body(parts_ref, hs2_ref, dinv_ref, w2_ref, b2_ref, out_ref):
    agg = parts_ref[0, :N] + parts_ref[1, :N] + hs2_ref[...]
    pre = dinv_ref[...] * agg
    logits = jnp.dot(pre, w2_ref[...],
                     preferred_element_type=jnp.float32) + b2_ref[...]
    m = jnp.max(logits, axis=1, keepdims=True)
    lse = m + jnp.log(jnp.sum(jnp.exp(logits - m), axis=1, keepdims=True))
    out_ref[...] = logits - lse


_tc3 = pl.pallas_call(
    _tc3_body,
    out_shape=jax.ShapeDtypeStruct((N, D_OUT), jnp.float32),
)


def kernel(x, edge_index, W1, b1, W2, b2):
    src = edge_index[0].astype(jnp.int32)
    dst = edge_index[1].astype(jnp.int32)
    pad = EW_PAD - EW
    srcp = jnp.pad(src.reshape(NW, EW), ((0, 0), (0, pad)),
                   constant_values=0).reshape(NW, NB, CB)
    dstp = jnp.pad(dst.reshape(NW, EW), ((0, 0), (0, pad)),
                   constant_values=N).reshape(NW, NB, CB)
    zeros16 = jnp.zeros((N_PAD, D_HID), jnp.float32)
    zeros_d = jnp.zeros((N_PAD, WD), jnp.float32)
    ones_d = jnp.ones((CB, WD), jnp.float32)

    deg_parts = _sc_deg(dstp, ones_d, zeros_d)
    dinv, hs1 = _tc1(deg_parts, x, W1)
    agg1 = _sc_agg(hs1, srcp, dstp, zeros16)
    hs2 = _tc2(agg1, hs1, dinv, b1)
    agg2 = _sc_agg(hs2, srcp, dstp, zeros16)
    return _tc3(agg2, hs2, dinv, W2, b2)
